# Initial kernel scaffold; baseline (speedup 1.0000x reference)
import jax
import jax.numpy as jnp
from jax.experimental import pallas as pl


def kernel(input_, table_ids):
    raise NotImplementedError("write your pallas kernel here")


# speedup vs baseline: 1.1027x; 1.1027x over previous
import functools

import jax
import jax.numpy as jnp
from jax import lax
from jax.experimental import pallas as pl
from jax.experimental.pallas import tpu as pltpu
from jax.experimental.pallas import tpu_sc as plsc

GRP = 128
K = 8


@functools.lru_cache(maxsize=None)
def _make_gather(B, V, D):
    info = plsc.get_sparse_core_info()
    NC, NS = info.num_cores, info.num_subcores
    NW = NC * NS
    assert B % (NW * GRP) == 0
    b_per_w = B // NW
    n_grp = b_per_w // GRP
    assert n_grp % K == 0
    n_burst = n_grp // K
    mesh = plsc.VectorSubcoreMesh(core_axis_name="c", subcore_axis_name="s")

    @functools.partial(
        pl.kernel,
        mesh=mesh,
        compiler_params=pltpu.CompilerParams(use_tc_tiling_on_sc=False),
        out_type=jax.ShapeDtypeStruct((B, D), jnp.float32),
        scratch_types=[
            pltpu.VMEM((n_grp, GRP), jnp.int32),
            pltpu.VMEM((K * GRP, D), jnp.float32),
            pltpu.SemaphoreType.DMA,
        ],
    )
    def gather_kernel(idx_hbm, table_hbm, out_hbm, idx_v, rows_v, gsem):
        wid = lax.axis_index("s") * NC + lax.axis_index("c")
        pltpu.sync_copy(idx_hbm.at[pl.ds(wid * n_grp, n_grp)], idx_v)

        def burst(b, _):
            for j in range(K):
                pltpu.async_copy(
                    table_hbm.at[idx_v.at[b * K + j]],
                    rows_v.at[pl.ds(j * GRP, GRP)],
                    gsem,
                )
            for j in range(K):
                pltpu.make_async_copy(
                    table_hbm.at[idx_v.at[b * K + j]],
                    rows_v.at[pl.ds(j * GRP, GRP)],
                    gsem,
                ).wait()
            out_base = pl.multiple_of(wid * b_per_w + b * (K * GRP), K * GRP)
            pltpu.sync_copy(rows_v, out_hbm.at[pl.ds(out_base, K * GRP)])
            return ()

        lax.fori_loop(0, n_burst, burst, (), unroll=False)

    return gather_kernel


def kernel(input_, table_ids):
    B0, H = input_.shape
    V, D = table_ids.shape
    B = B0 * H
    idx = input_.astype(jnp.int32).reshape(B // GRP, GRP)
    out = _make_gather(B, V, D)(idx, table_ids)
    return out.reshape(B0, H, D)


# speedup vs baseline: 1.1137x; 1.0100x over previous
import functools

import jax
import jax.numpy as jnp
from jax import lax
from jax.experimental import pallas as pl
from jax.experimental.pallas import tpu as pltpu
from jax.experimental.pallas import tpu_sc as plsc

GRP = 128
K = 8


@functools.lru_cache(maxsize=None)
def _make_gather(B, V, D):
    info = plsc.get_sparse_core_info()
    NC, NS = info.num_cores, info.num_subcores
    NW = NC * NS
    assert B % (NW * GRP) == 0
    b_per_w = B // NW
    n_grp = b_per_w // GRP
    assert n_grp % K == 0
    n_burst = n_grp // K
    mesh = plsc.VectorSubcoreMesh(core_axis_name="c", subcore_axis_name="s")

    @functools.partial(
        pl.kernel,
        mesh=mesh,
        compiler_params=pltpu.CompilerParams(use_tc_tiling_on_sc=False),
        out_type=jax.ShapeDtypeStruct((B, D), jnp.float32),
        scratch_types=[
            pltpu.VMEM((n_grp, GRP), jnp.int32),
            pltpu.VMEM((2, K * GRP, D), jnp.float32),
            pltpu.SemaphoreType.DMA((2,)),
        ],
    )
    def gather_kernel(idx_hbm, table_hbm, out_hbm, idx_v, rows_v, gsem):
        wid = lax.axis_index("s") * NC + lax.axis_index("c")
        pltpu.sync_copy(idx_hbm.at[pl.ds(wid * n_grp, n_grp)], idx_v)

        def fire(b, slot):
            for j in range(K):
                pltpu.async_copy(
                    table_hbm.at[idx_v.at[b * K + j]],
                    rows_v.at[slot].at[pl.ds(j * GRP, GRP)],
                    gsem.at[slot],
                )

        def drain_and_write(b, slot):
            for j in range(K):
                pltpu.make_async_copy(
                    table_hbm.at[idx_v.at[b * K + j]],
                    rows_v.at[slot].at[pl.ds(j * GRP, GRP)],
                    gsem.at[slot],
                ).wait()
            out_base = pl.multiple_of(wid * b_per_w + b * (K * GRP), K * GRP)
            pltpu.sync_copy(rows_v.at[slot], out_hbm.at[pl.ds(out_base, K * GRP)])

        fire(0, 0)

        def burst(b, _):
            fire(b + 1, lax.rem(b + 1, 2))
            drain_and_write(b, lax.rem(b, 2))
            return ()

        lax.fori_loop(0, n_burst - 1, burst, (), unroll=False)
        drain_and_write(n_burst - 1, (n_burst - 1) % 2)

    return gather_kernel


def kernel(input_, table_ids):
    B0, H = input_.shape
    V, D = table_ids.shape
    B = B0 * H
    idx = input_.astype(jnp.int32).reshape(B // GRP, GRP)
    out = _make_gather(B, V, D)(idx, table_ids)
    return out.reshape(B0, H, D)
